# baseline (speedup 1.0000x reference)
import functools

import jax
import jax.numpy as jnp
from jax.experimental import pallas as pl
from jax.experimental.pallas import tpu as pltpu

CHUNK = 64
BASE_LR = 0.01
MAX_GRAD_NORM = 1.0
EPS = 1e-6
_BF = jnp.bfloat16


def _dg(a, b, ca, cb):
    return jax.lax.dot_general(
        a.astype(_BF), b.astype(_BF),
        dimension_numbers=(((ca,), (cb,)), ((), ())),
        preferred_element_type=jnp.float32)


def _rms(x, w, eps=EPS):
    return x * jax.lax.rsqrt(jnp.mean(x * x, axis=-1, keepdims=True) + eps) * w


def _proj_body(x_ref, w2_ref, kn_ref, cw_ref,
               k_ref, v_ref, coef_ref, *, H, PC):
    xt = x_ref[0, 0]
    z = _dg(xt, w2_ref[...], 1, 0)
    k = _rms(jax.nn.silu(z[:, :H]), kn_ref[...])
    v = jax.nn.silu(z[:, H:])
    k_ref[...] = k.reshape(k_ref.shape).astype(_BF)
    v_ref[...] = v.reshape(v_ref.shape).astype(_BF)
    xt4 = xt.reshape(PC, CHUNK, H)
    rows = [jnp.stack([jnp.sum(xt4[l] * cw_ref[i]) for l in range(PC)])
            for i in range(3)]
    coef_ref[0, 0] = jnp.stack(rows)


def _scan_body(k_ref, v_ref, lin_ref, lni_ref, w0i_ref, w1i_ref,
               w0o_ref, w1o_ref, lno_ref,
               w0_s, w1_s, s0_s, s1_s, hb_s, sd_s, g0_s, g1_s,
               y_s, dyb_s, ln_s, sln_s,
               sem0, sem1, *, B, H, NC, NTOT, IC, ICW):
    i = pl.program_id(0)
    ROWS = B * CHUNK

    @pl.when(i == 0)
    def _init():
        cps = []
        for c in range(IC):
            cps.append(pltpu.make_async_copy(
                w0i_ref.at[:, c * ICW:(c + 1) * ICW], w0_s.at[c], sem0))
            cps.append(pltpu.make_async_copy(
                w1i_ref.at[c * ICW:(c + 1) * ICW, :], w1_s.at[c], sem1))
        for cp in cps:
            cp.start()
        s0_s[...] = jnp.zeros_like(s0_s)
        s1_s[...] = jnp.zeros_like(s1_s)
        sln_s[...] = jnp.zeros_like(sln_s)
        ln_s[...] = lni_ref[...]
        for cp in cps:
            cp.wait()

    lin = lin_ref[0]
    alpha = jax.nn.sigmoid(jnp.mean(lin[0]))
    beta = 1.0 - alpha
    eta = jax.nn.sigmoid(jnp.mean(lin[2]))
    th = jax.nn.sigmoid(lin[1]) * BASE_LR

    bid = jax.lax.broadcasted_iota(jnp.int32, (ROWS, B), 0) // CHUNK
    sel = bid == jax.lax.broadcasted_iota(jnp.int32, (ROWS, B), 1)
    th_col = jnp.sum(jnp.where(sel, th.reshape(1, B), 0.0),
                     axis=1, keepdims=True)

    y_s[...] = jnp.zeros_like(y_s)

    def _fwd(c, carry):
        z0c = _dg(k_ref[0], w0_s[c], 1, 0)
        sg = jax.nn.sigmoid(z0c)
        hc = z0c * sg
        hb_s[c] = hc.astype(_BF)
        sd_s[c] = (sg * (1.0 + z0c * (1.0 - sg))).astype(_BF)
        y_s[...] += _dg(hc, w1_s[c], 1, 0)
        return carry
    jax.lax.fori_loop(0, IC, _fwd, 0)

    y = y_s[...]
    inv = jax.lax.rsqrt(jnp.mean(y * y, axis=-1, keepdims=True) + EPS)
    lnv = ln_s[...]
    r = k_ref[0].astype(jnp.float32) + y * inv * lnv \
        - v_ref[0].astype(jnp.float32)
    G = (2.0 / NTOT) * th_col * r
    Gy = G * y
    g_ln = jnp.sum(Gy * inv, axis=0, keepdims=True)
    rowdot = jnp.sum(Gy * lnv, axis=1, keepdims=True)
    dY = inv * (G * lnv) - (inv * inv * inv) * (rowdot / H) * y
    dyb_s[...] = dY.astype(_BF)

    def _bwd(c, tot):
        dyb = dyb_s[...]
        g1c = _dg(hb_s[c], dyb, 0, 0)
        g1_s[c] = g1c.astype(_BF)
        tot = tot + jnp.sum(g1c * g1c)
        dhc = _dg(dyb, w1_s[c], 1, 1)
        dZc = dhc * sd_s[c].astype(jnp.float32)
        g0c = _dg(k_ref[0], dZc, 0, 0)
        g0_s[c] = g0c.astype(_BF)
        return tot + jnp.sum(g0c * g0c)
    tot = jax.lax.fori_loop(0, IC, _bwd, jnp.sum(g_ln * g_ln))
    clip = jnp.minimum(MAX_GRAD_NORM / (jnp.sqrt(tot) + 1e-6), 1.0)

    def _upd(c, carry):
        s0_s[c] = eta * s0_s[c] - clip * g0_s[c].astype(jnp.float32)
        w0_s[c] = beta * w0_s[c] + s0_s[c]
        s1_s[c] = eta * s1_s[c] - clip * g1_s[c].astype(jnp.float32)
        w1_s[c] = beta * w1_s[c] + s1_s[c]
        return carry
    jax.lax.fori_loop(0, IC, _upd, 0)
    sln_s[...] = eta * sln_s[...] - clip * g_ln
    ln_s[...] = beta * ln_s[...] + sln_s[...]

    @pl.when(i == NC - 1)
    def _fin():
        for c in range(IC):
            g0_s[c] = w0_s[c].astype(_BF)
            g1_s[c] = w1_s[c].astype(_BF)
            cp0 = pltpu.make_async_copy(
                g0_s.at[c], w0o_ref.at[:, c * ICW:(c + 1) * ICW], sem0)
            cp1 = pltpu.make_async_copy(
                g1_s.at[c], w1o_ref.at[c * ICW:(c + 1) * ICW, :], sem1)
            cp0.start()
            cp1.start()
            cp0.wait()
            cp1.wait()
        lno_ref[...] = ln_s[...]


def _retr_body(x_ref, wq_ref, qn_ref, w0_ref, w1_ref, lnf_ref, o_ref):
    xt = x_ref[...]
    q = _rms(jax.nn.silu(_dg(xt, wq_ref[...], 1, 0)), qn_ref[...])
    h = jax.nn.silu(_dg(q, w0_ref[...], 1, 0))
    y = _dg(h, w1_ref[...], 1, 0)
    o_ref[...] = q + _rms(y, lnf_ref[...])


def kernel(x, wq_w, wk_w, wv_w, q_norm_w, k_norm_w, alpha_w, theta_w, eta_w,
           mem_w0, mem_w1, mem_ln_w):
    B, S, H = x.shape
    INTER = mem_w0.shape[0]
    NC = S // CHUNK
    ROWS = B * CHUNK
    NTOT = B * CHUNK * H
    PC = min(16, NC)
    NCT = NC // PC
    PR = PC * CHUNK

    w2t = jnp.concatenate([wk_w, wv_w], axis=0).T.astype(_BF)
    cw = jnp.concatenate([alpha_w, theta_w, eta_w], axis=0)
    cw = cw.reshape(3, CHUNK, H)
    qn = q_norm_w.reshape(1, H)
    kn = k_norm_w.reshape(1, H)
    ln0 = mem_ln_w.reshape(1, H)

    k_arr, v_arr, coef = pl.pallas_call(
        functools.partial(_proj_body, H=H, PC=PC),
        grid=(B, NCT),
        in_specs=[
            pl.BlockSpec((1, 1, PR, H), lambda b, j: (b, j, 0, 0)),
            pl.BlockSpec((H, 2 * H), lambda b, j: (0, 0)),
            pl.BlockSpec((1, H), lambda b, j: (0, 0)),
            pl.BlockSpec((3, CHUNK, H), lambda b, j: (0, 0, 0)),
        ],
        out_specs=[
            pl.BlockSpec((PC, CHUNK, H), lambda b, j: (j, b, 0)),
            pl.BlockSpec((PC, CHUNK, H), lambda b, j: (j, b, 0)),
            pl.BlockSpec((1, 1, 3, PC), lambda b, j: (b, j, 0, 0)),
        ],
        out_shape=[
            jax.ShapeDtypeStruct((NC, ROWS, H), _BF),
            jax.ShapeDtypeStruct((NC, ROWS, H), _BF),
            jax.ShapeDtypeStruct((B, NCT, 3, PC), jnp.float32),
        ],
        compiler_params=pltpu.CompilerParams(
            dimension_semantics=("parallel", "arbitrary"),
            vmem_limit_bytes=56 * 1024 * 1024,
        ),
        name="ttt_proj",
    )(x.reshape(B, NCT, PR, H), w2t, kn, cw)

    lin = coef.transpose(1, 3, 2, 0).reshape(NC, 3, B)

    ICW = min(1024, INTER)
    IC = INTER // ICW
    w0b, w1b, ln_f = pl.pallas_call(
        functools.partial(_scan_body, B=B, H=H, NC=NC, NTOT=NTOT,
                          IC=IC, ICW=ICW),
        grid=(NC,),
        in_specs=[
            pl.BlockSpec((1, ROWS, H), lambda i: (i, 0, 0)),
            pl.BlockSpec((1, ROWS, H), lambda i: (i, 0, 0)),
            pl.BlockSpec((1, 3, B), lambda i: (i, 0, 0)),
            pl.BlockSpec((1, H), lambda i: (0, 0)),
            pl.BlockSpec(memory_space=pl.ANY),
            pl.BlockSpec(memory_space=pl.ANY),
        ],
        out_specs=[
            pl.BlockSpec(memory_space=pl.ANY),
            pl.BlockSpec(memory_space=pl.ANY),
            pl.BlockSpec((1, H), lambda i: (0, 0)),
        ],
        out_shape=[
            jax.ShapeDtypeStruct((H, INTER), _BF),
            jax.ShapeDtypeStruct((INTER, H), _BF),
            jax.ShapeDtypeStruct((1, H), jnp.float32),
        ],
        scratch_shapes=[
            pltpu.VMEM((IC, H, ICW), jnp.float32),
            pltpu.VMEM((IC, ICW, H), jnp.float32),
            pltpu.VMEM((IC, H, ICW), jnp.float32),
            pltpu.VMEM((IC, ICW, H), jnp.float32),
            pltpu.VMEM((IC, ROWS, ICW), _BF),
            pltpu.VMEM((IC, ROWS, ICW), _BF),
            pltpu.VMEM((IC, H, ICW), _BF),
            pltpu.VMEM((IC, ICW, H), _BF),
            pltpu.VMEM((ROWS, H), jnp.float32),
            pltpu.VMEM((ROWS, H), _BF),
            pltpu.VMEM((1, H), jnp.float32),
            pltpu.VMEM((1, H), jnp.float32),
            pltpu.SemaphoreType.DMA,
            pltpu.SemaphoreType.DMA,
        ],
        compiler_params=pltpu.CompilerParams(
            dimension_semantics=("arbitrary",),
            vmem_limit_bytes=58 * 1024 * 1024,
        ),
        name="ttt_scan",
    )(k_arr, v_arr, lin, ln0, mem_w0.T, mem_w1.T)

    TC = min(1024, B * S)
    T = (B * S) // TC
    out = pl.pallas_call(
        _retr_body,
        grid=(T,),
        in_specs=[
            pl.BlockSpec((TC, H), lambda t: (t, 0)),
            pl.BlockSpec((H, H), lambda t: (0, 0)),
            pl.BlockSpec((1, H), lambda t: (0, 0)),
            pl.BlockSpec((H, INTER), lambda t: (0, 0)),
            pl.BlockSpec((INTER, H), lambda t: (0, 0)),
            pl.BlockSpec((1, H), lambda t: (0, 0)),
        ],
        out_specs=pl.BlockSpec((TC, H), lambda t: (t, 0)),
        out_shape=jax.ShapeDtypeStruct((B * S, H), jnp.float32),
        compiler_params=pltpu.CompilerParams(
            dimension_semantics=("parallel",),
            vmem_limit_bytes=56 * 1024 * 1024,
        ),
        name="ttt_retrieve",
    )(x.reshape(B * S, H), wq_w.T.astype(_BF), qn, w0b, w1b, ln_f)

    return out.reshape(B, S, H)


# speedup vs baseline: 1.0260x; 1.0260x over previous
import functools

import jax
import jax.numpy as jnp
from jax.experimental import pallas as pl
from jax.experimental.pallas import tpu as pltpu

CHUNK = 64
BASE_LR = 0.01
MAX_GRAD_NORM = 1.0
EPS = 1e-6
_BF = jnp.bfloat16


def _dg(a, b, ca, cb):
    return jax.lax.dot_general(
        a.astype(_BF), b.astype(_BF),
        dimension_numbers=(((ca,), (cb,)), ((), ())),
        preferred_element_type=jnp.float32)


def _rms(x, w, eps=EPS):
    return x * jax.lax.rsqrt(jnp.mean(x * x, axis=-1, keepdims=True) + eps) * w


def _proj_body(x_ref, w2_ref, kn_ref, cw_ref,
               k_ref, v_ref, coef_ref, *, H, PC):
    xt = x_ref[0, 0]
    z = _dg(xt, w2_ref[...], 1, 0)
    k = _rms(jax.nn.silu(z[:, :H]), kn_ref[...])
    v = jax.nn.silu(z[:, H:])
    k_ref[...] = k.reshape(k_ref.shape).astype(_BF)
    v_ref[...] = v.reshape(v_ref.shape).astype(_BF)
    xt4 = xt.reshape(PC, CHUNK, H)
    rows = [jnp.stack([jnp.sum(xt4[l] * cw_ref[i]) for l in range(PC)])
            for i in range(3)]
    coef_ref[0, 0] = jnp.stack(rows)


def _scan_body(k_ref, v_ref, lin_ref, lni_ref, w0i_ref, w1i_ref,
               w0o_ref, w1o_ref, lno_ref,
               w0_s, w1_s, s0_s, s1_s, hb_s, sd_s, g0_s, g1_s,
               dyb_s, ln_s, sln_s,
               sem0, sem1, *, B, H, NC, NTOT, IC, ICW):
    i = pl.program_id(0)
    ROWS = B * CHUNK

    @pl.when(i == 0)
    def _init():
        cps = []
        for c in range(IC):
            cps.append(pltpu.make_async_copy(
                w0i_ref.at[:, c * ICW:(c + 1) * ICW], w0_s.at[c], sem0))
            cps.append(pltpu.make_async_copy(
                w1i_ref.at[c * ICW:(c + 1) * ICW, :], w1_s.at[c], sem1))
        for cp in cps:
            cp.start()
        s0_s[...] = jnp.zeros_like(s0_s)
        s1_s[...] = jnp.zeros_like(s1_s)
        sln_s[...] = jnp.zeros_like(sln_s)
        ln_s[...] = lni_ref[...]
        for cp in cps:
            cp.wait()

    lin = lin_ref[0]
    alpha = jax.nn.sigmoid(jnp.mean(lin[0]))
    beta = 1.0 - alpha
    eta = jax.nn.sigmoid(jnp.mean(lin[2]))
    th = jax.nn.sigmoid(lin[1]) * BASE_LR

    bid = jax.lax.broadcasted_iota(jnp.int32, (ROWS, B), 0) // CHUNK
    sel = bid == jax.lax.broadcasted_iota(jnp.int32, (ROWS, B), 1)
    th_col = jnp.sum(jnp.where(sel, th.reshape(1, B), 0.0),
                     axis=1, keepdims=True)

    y = None
    for c in range(IC):
        z0c = _dg(k_ref[0], w0_s[c], 1, 0)
        sg = jax.nn.sigmoid(z0c)
        hc = z0c * sg
        hb_s[c] = hc.astype(_BF)
        sd_s[c] = (sg * (1.0 + z0c * (1.0 - sg))).astype(_BF)
        yc = _dg(hc, w1_s[c], 1, 0)
        y = yc if y is None else y + yc
    inv = jax.lax.rsqrt(jnp.mean(y * y, axis=-1, keepdims=True) + EPS)
    lnv = ln_s[...]
    r = k_ref[0].astype(jnp.float32) + y * inv * lnv \
        - v_ref[0].astype(jnp.float32)
    G = (2.0 / NTOT) * th_col * r
    Gy = G * y
    g_ln = jnp.sum(Gy * inv, axis=0, keepdims=True)
    rowdot = jnp.sum(Gy * lnv, axis=1, keepdims=True)
    dY = inv * (G * lnv) - (inv * inv * inv) * (rowdot / H) * y
    dyb_s[...] = dY.astype(_BF)

    def _bwd(c, tot):
        dyb = dyb_s[...]
        g1c = _dg(hb_s[c], dyb, 0, 0)
        g1_s[c] = g1c.astype(_BF)
        tot = tot + jnp.sum(g1c * g1c)
        dhc = _dg(dyb, w1_s[c], 1, 1)
        dZc = dhc * sd_s[c].astype(jnp.float32)
        g0c = _dg(k_ref[0], dZc, 0, 0)
        g0_s[c] = g0c.astype(_BF)
        return tot + jnp.sum(g0c * g0c)
    tot = jax.lax.fori_loop(0, IC, _bwd, jnp.sum(g_ln * g_ln))
    clip = jnp.minimum(MAX_GRAD_NORM / (jnp.sqrt(tot) + 1e-6), 1.0)

    def _upd(c, carry):
        s0_s[c] = eta * s0_s[c] - clip * g0_s[c].astype(jnp.float32)
        w0_s[c] = beta * w0_s[c] + s0_s[c]
        s1_s[c] = eta * s1_s[c] - clip * g1_s[c].astype(jnp.float32)
        w1_s[c] = beta * w1_s[c] + s1_s[c]
        return carry
    jax.lax.fori_loop(0, IC, _upd, 0)
    sln_s[...] = eta * sln_s[...] - clip * g_ln
    ln_s[...] = beta * ln_s[...] + sln_s[...]

    @pl.when(i == NC - 1)
    def _fin():
        for c in range(IC):
            g0_s[c] = w0_s[c].astype(_BF)
            g1_s[c] = w1_s[c].astype(_BF)
            cp0 = pltpu.make_async_copy(
                g0_s.at[c], w0o_ref.at[:, c * ICW:(c + 1) * ICW], sem0)
            cp1 = pltpu.make_async_copy(
                g1_s.at[c], w1o_ref.at[c * ICW:(c + 1) * ICW, :], sem1)
            cp0.start()
            cp1.start()
            cp0.wait()
            cp1.wait()
        lno_ref[...] = ln_s[...]


def _retr_body(x_ref, wq_ref, qn_ref, w0_ref, w1_ref, lnf_ref, o_ref):
    xt = x_ref[...]
    q = _rms(jax.nn.silu(_dg(xt, wq_ref[...], 1, 0)), qn_ref[...])
    h = jax.nn.silu(_dg(q, w0_ref[...], 1, 0))
    y = _dg(h, w1_ref[...], 1, 0)
    o_ref[...] = q + _rms(y, lnf_ref[...])


def kernel(x, wq_w, wk_w, wv_w, q_norm_w, k_norm_w, alpha_w, theta_w, eta_w,
           mem_w0, mem_w1, mem_ln_w):
    B, S, H = x.shape
    INTER = mem_w0.shape[0]
    NC = S // CHUNK
    ROWS = B * CHUNK
    NTOT = B * CHUNK * H
    PC = min(8, NC)
    NCT = NC // PC
    PR = PC * CHUNK

    w2t = jnp.concatenate([wk_w, wv_w], axis=0).T.astype(_BF)
    cw = jnp.concatenate([alpha_w, theta_w, eta_w], axis=0)
    cw = cw.reshape(3, CHUNK, H)
    qn = q_norm_w.reshape(1, H)
    kn = k_norm_w.reshape(1, H)
    ln0 = mem_ln_w.reshape(1, H)

    k_arr, v_arr, coef = pl.pallas_call(
        functools.partial(_proj_body, H=H, PC=PC),
        grid=(B, NCT),
        in_specs=[
            pl.BlockSpec((1, 1, PR, H), lambda b, j: (b, j, 0, 0)),
            pl.BlockSpec((H, 2 * H), lambda b, j: (0, 0)),
            pl.BlockSpec((1, H), lambda b, j: (0, 0)),
            pl.BlockSpec((3, CHUNK, H), lambda b, j: (0, 0, 0)),
        ],
        out_specs=[
            pl.BlockSpec((PC, CHUNK, H), lambda b, j: (j, b, 0)),
            pl.BlockSpec((PC, CHUNK, H), lambda b, j: (j, b, 0)),
            pl.BlockSpec((1, 1, 3, PC), lambda b, j: (b, j, 0, 0)),
        ],
        out_shape=[
            jax.ShapeDtypeStruct((NC, ROWS, H), _BF),
            jax.ShapeDtypeStruct((NC, ROWS, H), _BF),
            jax.ShapeDtypeStruct((B, NCT, 3, PC), jnp.float32),
        ],
        compiler_params=pltpu.CompilerParams(
            dimension_semantics=("parallel", "arbitrary"),
            vmem_limit_bytes=56 * 1024 * 1024,
        ),
        name="ttt_proj",
    )(x.reshape(B, NCT, PR, H), w2t, kn, cw)

    lin = coef.transpose(1, 3, 2, 0).reshape(NC, 3, B)

    ICW = min(1024, INTER)
    IC = INTER // ICW
    w0b, w1b, ln_f = pl.pallas_call(
        functools.partial(_scan_body, B=B, H=H, NC=NC, NTOT=NTOT,
                          IC=IC, ICW=ICW),
        grid=(NC,),
        in_specs=[
            pl.BlockSpec((1, ROWS, H), lambda i: (i, 0, 0)),
            pl.BlockSpec((1, ROWS, H), lambda i: (i, 0, 0)),
            pl.BlockSpec((1, 3, B), lambda i: (i, 0, 0)),
            pl.BlockSpec((1, H), lambda i: (0, 0)),
            pl.BlockSpec(memory_space=pl.ANY),
            pl.BlockSpec(memory_space=pl.ANY),
        ],
        out_specs=[
            pl.BlockSpec(memory_space=pl.ANY),
            pl.BlockSpec(memory_space=pl.ANY),
            pl.BlockSpec((1, H), lambda i: (0, 0)),
        ],
        out_shape=[
            jax.ShapeDtypeStruct((H, INTER), _BF),
            jax.ShapeDtypeStruct((INTER, H), _BF),
            jax.ShapeDtypeStruct((1, H), jnp.float32),
        ],
        scratch_shapes=[
            pltpu.VMEM((IC, H, ICW), jnp.float32),
            pltpu.VMEM((IC, ICW, H), jnp.float32),
            pltpu.VMEM((IC, H, ICW), jnp.float32),
            pltpu.VMEM((IC, ICW, H), jnp.float32),
            pltpu.VMEM((IC, ROWS, ICW), _BF),
            pltpu.VMEM((IC, ROWS, ICW), _BF),
            pltpu.VMEM((IC, H, ICW), _BF),
            pltpu.VMEM((IC, ICW, H), _BF),
            pltpu.VMEM((ROWS, H), _BF),
            pltpu.VMEM((1, H), jnp.float32),
            pltpu.VMEM((1, H), jnp.float32),
            pltpu.SemaphoreType.DMA,
            pltpu.SemaphoreType.DMA,
        ],
        compiler_params=pltpu.CompilerParams(
            dimension_semantics=("arbitrary",),
            vmem_limit_bytes=58 * 1024 * 1024,
        ),
        name="ttt_scan",
    )(k_arr, v_arr, lin, ln0, mem_w0.T, mem_w1.T)

    TC = min(1024, B * S)
    T = (B * S) // TC
    out = pl.pallas_call(
        _retr_body,
        grid=(T,),
        in_specs=[
            pl.BlockSpec((TC, H), lambda t: (t, 0)),
            pl.BlockSpec((H, H), lambda t: (0, 0)),
            pl.BlockSpec((1, H), lambda t: (0, 0)),
            pl.BlockSpec((H, INTER), lambda t: (0, 0)),
            pl.BlockSpec((INTER, H), lambda t: (0, 0)),
            pl.BlockSpec((1, H), lambda t: (0, 0)),
        ],
        out_specs=pl.BlockSpec((TC, H), lambda t: (t, 0)),
        out_shape=jax.ShapeDtypeStruct((B * S, H), jnp.float32),
        compiler_params=pltpu.CompilerParams(
            dimension_semantics=("parallel",),
            vmem_limit_bytes=56 * 1024 * 1024,
        ),
        name="ttt_retrieve",
    )(x.reshape(B * S, H), wq_w.T.astype(_BF), qn, w0b, w1b, ln_f)

    return out.reshape(B, S, H)
